# baseline (speedup 1.0000x reference)
import functools

import jax
import jax.numpy as jnp
from jax import lax
from jax.experimental import pallas as pl
from jax.experimental.pallas import tpu as pltpu
from jax.experimental.pallas import tpu_sc as plsc

N = 10000
E = 320000
D_IN = 128
HID = 128
NCLS = 64
ALPHA = 0.1
KPROP = 10
EPS = 1e-5

NTILES = 16
LANES = 128
CH = 160
IDXB = 16
EP = NTILES * CH * LANES
NP = 10240
RPT = NP // NTILES


def _deg_sc(dst3, ones_rpt):
    mesh = plsc.VectorSubcoreMesh(core_axis_name="c", subcore_axis_name="s")

    @functools.partial(
        pl.kernel,
        out_type=jax.ShapeDtypeStruct((NP, 16), jnp.float32),
        mesh=mesh,
        scratch_types=[
            pltpu.VMEM_SHARED((NP, 16), jnp.float32),
            pltpu.VMEM((CH, LANES), jnp.int32),
            pltpu.VMEM((RPT, 16), jnp.float32),
            pltpu.VMEM((RPT, 16), jnp.float32),
        ],
        compiler_params=pltpu.CompilerParams(use_tc_tiling_on_sc=False),
    )
    def body(dst_r, ones_r, out_r, deg_s, dstb, onesb, degc):
        cid = lax.axis_index("c")
        t = lax.axis_index("s")
        r0 = t * RPT
        pltpu.sync_copy(ones_r, onesb)
        pltpu.sync_copy(dst_r.at[t], dstb)
        pltpu.sync_copy(onesb, deg_s.at[pl.ds(r0, RPT)])
        plsc.subcore_barrier()

        def chunk(j, carry):
            pltpu.sync_copy(onesb.at[pl.ds(0, LANES)], deg_s.at[dstb.at[j]],
                            add=True)
            return carry

        lax.fori_loop(0, CH, chunk, None)
        plsc.subcore_barrier()

        @pl.when(cid == 0)
        def _():
            pltpu.sync_copy(deg_s.at[pl.ds(r0, RPT)], degc)
            pltpu.sync_copy(degc, out_r.at[pl.ds(r0, RPT)])

    return body(dst3, ones_rpt)


def _appnp_sc(src3, dst3, xda, d2a, sqb, zrows, F):
    mesh = plsc.VectorSubcoreMesh(core_axis_name="c", subcore_axis_name="s")
    CB = 64

    @functools.partial(
        pl.kernel,
        out_type=(
            jax.ShapeDtypeStruct((2, NP, F), jnp.float32),
            jax.ShapeDtypeStruct((2, NP, F), jnp.float32),
        ),
        mesh=mesh,
        scratch_types=[
            pltpu.VMEM_SHARED((NP, F), jnp.float32),
            pltpu.VMEM_SHARED((NP, F), jnp.float32),
            pltpu.VMEM((IDXB, LANES), jnp.int32),
            pltpu.VMEM((IDXB, LANES), jnp.int32),
            pltpu.VMEM((2, LANES, F), jnp.float32),
            pltpu.VMEM((CB, F), jnp.float32),
            pltpu.VMEM((CB, F), jnp.float32),
            pltpu.VMEM((CB, F), jnp.float32),
            pltpu.VMEM((CB, 16), jnp.float32),
            pltpu.VMEM((CB, 16), jnp.float32),
            pltpu.VMEM((CB, F), jnp.float32),
            pltpu.SemaphoreType.DMA((2,)),
            pltpu.SemaphoreType.DMA((2,)),
        ],
        compiler_params=pltpu.CompilerParams(use_tc_tiling_on_sc=False),
    )
    def body(src_r, dst_r, xd_r, d2_r, sq_r, z_r, out_r, u_r,
             T, xdS, srcb, dstb, gb, cT, cU, cX, cd, cs, zT, gs, ss):
        cid = lax.axis_index("c")
        t = lax.axis_index("s")
        r0 = t * RPT
        Uc = u_r.at[cid]

        pltpu.sync_copy(z_r, zT)
        for cc in range(RPT // CB):
            rr = r0 + cc * CB
            pltpu.sync_copy(xd_r.at[cid, pl.ds(rr, CB)], cX)
            pltpu.sync_copy(cX, xdS.at[pl.ds(rr, CB)])

            def urow(rI, carry):
                for c in range(F // 16):
                    sl = pl.ds(c * 16, 16)
                    cU[rI, sl] = cX[rI, sl] * (1.0 / ALPHA)
                return carry

            lax.fori_loop(0, CB, urow, None)
            pltpu.sync_copy(cU, Uc.at[pl.ds(rr, CB)])
            pltpu.sync_copy(zT, T.at[pl.ds(rr, CB)])
        plsc.subcore_barrier()

        def edge_phase():
            def drain(sem, p):
                pltpu.make_async_copy(xd_r.at[cid, pl.ds(0, LANES)],
                                      gb.at[p], sem.at[p]).wait()

            def blk(bI, carry):
                pltpu.sync_copy(src_r.at[t, pl.ds(bI * IDXB, IDXB)], srcb)
                pltpu.sync_copy(dst_r.at[t, pl.ds(bI * IDXB, IDXB)], dstb)
                pltpu.async_copy(Uc.at[srcb.at[0]], gb.at[0], gs.at[0])

                def chunk(j, c2):
                    p = lax.rem(j, 2)
                    pn = lax.rem(j + 1, 2)

                    @pl.when(j >= 1)
                    def _():
                        drain(ss, pn)

                    @pl.when(j + 1 < IDXB)
                    def _():
                        pltpu.async_copy(Uc.at[srcb.at[j + 1]], gb.at[pn],
                                         gs.at[pn])

                    drain(gs, p)
                    pltpu.async_copy(gb.at[p], T.at[dstb.at[j]], ss.at[p],
                                     add=True)
                    return c2

                lax.fori_loop(0, IDXB, chunk, None)
                drain(ss, 1)
                return carry

            lax.fori_loop(0, CH // IDXB, blk, None)
            plsc.subcore_barrier()

        def combine(is_final):
            for cc in range(RPT // CB):
                rr = r0 + cc * CB
                pltpu.sync_copy(T.at[pl.ds(rr, CB)], cT)
                pltpu.sync_copy(Uc.at[pl.ds(rr, CB)], cU)
                pltpu.sync_copy(xdS.at[pl.ds(rr, CB)], cX)
                pltpu.sync_copy(d2_r.at[pl.ds(rr, CB)], cd)
                if is_final:
                    pltpu.sync_copy(sq_r.at[pl.ds(rr, CB)], cs)

                def row(rI, carry):
                    bd = cd[rI, pl.ds(0, 16)]
                    for c in range(F // 16):
                        sl = pl.ds(c * 16, 16)
                        un = bd * (cT[rI, sl] + cU[rI, sl]) + cX[rI, sl]
                        if is_final:
                            un = un * cs[rI, pl.ds(0, 16)]
                        cU[rI, sl] = un
                    return carry

                lax.fori_loop(0, CB, row, None)
                if is_final:
                    pltpu.sync_copy(cU, out_r.at[cid, pl.ds(rr, CB)])
                else:
                    pltpu.sync_copy(cU, Uc.at[pl.ds(rr, CB)])
                    pltpu.sync_copy(zT, T.at[pl.ds(rr, CB)])

        def round_body(k, carry):
            edge_phase()
            combine(False)
            plsc.subcore_barrier()
            return carry

        lax.fori_loop(0, KPROP - 1, round_body, None)
        edge_phase()
        combine(True)

    return body(src3, dst3, xda, d2a, sqb, zrows)[0]


def _tc_prep(x_p, W1, b1, deg2):
    def body(x_r, w_r, b_r, deg_r, xd_r, d2_r, sq_r):
        deg = deg_r[:, 0:1]
        dis = lax.rsqrt(deg)
        h = jnp.dot(x_r[...], w_r[...].T,
                    preferred_element_type=jnp.float32) + b_r[...][None, :]
        mask = lax.broadcasted_iota(jnp.int32, (NP, 1), 0) < N
        hd = jnp.where(mask, (ALPHA * dis) * h, 0.0)
        xd_r[0] = hd[:, :HID // 2]
        xd_r[1] = hd[:, HID // 2:]
        d2_r[...] = jnp.broadcast_to((1.0 - ALPHA) / deg, (NP, 16))
        sq_r[...] = jnp.broadcast_to(jnp.sqrt(deg), (NP, 16))

    return pl.pallas_call(
        body,
        out_shape=[
            jax.ShapeDtypeStruct((2, NP, HID // 2), jnp.float32),
            jax.ShapeDtypeStruct((NP, 16), jnp.float32),
            jax.ShapeDtypeStruct((NP, 16), jnp.float32),
        ],
    )(x_p, W1, b1, deg2)


def _tc_mid(H, g, be, W, b, d2b, sqb, Fo):
    def body(H_r, g_r, be_r, w_r, b_r, d2_r, sq_r, out_r):
        h = jnp.concatenate([H_r[0], H_r[1]], axis=1)
        mask = lax.broadcasted_iota(jnp.int32, (NP, 1), 0) < N
        hm = jnp.where(mask, h, 0.0)
        m = jnp.sum(hm, axis=0, keepdims=True) / N
        dcen = jnp.where(mask, h - m, 0.0)
        v = jnp.sum(dcen * dcen, axis=0, keepdims=True) / N
        hn = g_r[...][None, :] * (h - m) * lax.rsqrt(v + EPS) + be_r[...][None, :]
        hrelu = jnp.maximum(hn, 0.0)
        h2 = jnp.dot(hrelu, w_r[...].T,
                     preferred_element_type=jnp.float32) + b_r[...][None, :]
        adis = (ALPHA / (1.0 - ALPHA)) * d2_r[:, 0:1] * sq_r[:, 0:1]
        xd = jnp.where(mask, adis * h2, 0.0)
        out_r[0] = xd[:, :Fo // 2]
        out_r[1] = xd[:, Fo // 2:]

    return pl.pallas_call(
        body,
        out_shape=jax.ShapeDtypeStruct((2, NP, Fo // 2), jnp.float32),
    )(H, g, be, W, b, d2b, sqb)


def _tc_final(H, g, be):
    def body(H_r, g_r, be_r, out_r):
        h = jnp.concatenate([H_r[0], H_r[1]], axis=1)
        mask = lax.broadcasted_iota(jnp.int32, (NP, 1), 0) < N
        hm = jnp.where(mask, h, 0.0)
        m = jnp.sum(hm, axis=0, keepdims=True) / N
        dcen = jnp.where(mask, h - m, 0.0)
        v = jnp.sum(dcen * dcen, axis=0, keepdims=True) / N
        out_r[...] = g_r[...][None, :] * (h - m) * lax.rsqrt(v + EPS) + be_r[...][None, :]

    return pl.pallas_call(
        body,
        out_shape=jax.ShapeDtypeStruct((NP, NCLS), jnp.float32),
    )(H, g, be)


def kernel(x, edge_index, W1, b1, Wx, bx, W2, b2, g1, be1, g3, be3, g2, be2):
    ei = edge_index.astype(jnp.int32)
    src = ei[0]
    dst = ei[1]
    pad = EP - E
    src3 = jnp.concatenate([src, jnp.zeros((pad,), jnp.int32)]).reshape(NTILES, CH, LANES)
    dst3 = jnp.concatenate([dst, jnp.full((pad,), N, jnp.int32)]).reshape(NTILES, CH, LANES)
    x_p = jnp.pad(x, ((0, NP - N), (0, 0)))
    ones_rpt = jnp.ones((RPT, 16), jnp.float32)
    z64 = jnp.zeros((64, HID // 2), jnp.float32)
    z32 = jnp.zeros((64, NCLS // 2), jnp.float32)

    deg2 = _deg_sc(dst3, ones_rpt)
    xd1, d2b, sqb = _tc_prep(x_p, W1, b1, deg2)
    H1 = _appnp_sc(src3, dst3, xd1, d2b, sqb, z64, HID // 2)
    xd2 = _tc_mid(H1, g1, be1, Wx, bx, d2b, sqb, HID)
    H2 = _appnp_sc(src3, dst3, xd2, d2b, sqb, z64, HID // 2)
    xd3 = _tc_mid(H2, g3, be3, W2, b2, d2b, sqb, NCLS)
    H3 = _appnp_sc(src3, dst3, xd3, d2b, sqb, z32, NCLS // 2)
    out = _tc_final(H3, g2, be2)
    return out[:N]


# speedup vs baseline: 1.8842x; 1.8842x over previous
import functools

import jax
import jax.numpy as jnp
from jax import lax
from jax.experimental import pallas as pl
from jax.experimental.pallas import tpu as pltpu
from jax.experimental.pallas import tpu_sc as plsc

N = 10000
E = 320000
D_IN = 128
HID = 128
NCLS = 64
ALPHA = 0.1
KPROP = 10
EPS = 1e-5

NTILES = 16
LANES = 128
CH = 160
IDXB = 16
EP = NTILES * CH * LANES
NP = 10240
RPT = NP // NTILES


def _deg_sc(dst3, ones_rpt):
    mesh = plsc.VectorSubcoreMesh(core_axis_name="c", subcore_axis_name="s")

    @functools.partial(
        pl.kernel,
        out_type=jax.ShapeDtypeStruct((NP, 16), jnp.float32),
        mesh=mesh,
        scratch_types=[
            pltpu.VMEM_SHARED((NP, 16), jnp.float32),
            pltpu.VMEM((CH, LANES), jnp.int32),
            pltpu.VMEM((RPT, 16), jnp.float32),
            pltpu.VMEM((RPT, 16), jnp.float32),
        ],
        compiler_params=pltpu.CompilerParams(use_tc_tiling_on_sc=False),
    )
    def body(dst_r, ones_r, out_r, deg_s, dstb, onesb, degc):
        cid = lax.axis_index("c")
        t = lax.axis_index("s")
        r0 = t * RPT
        pltpu.sync_copy(ones_r, onesb)
        pltpu.sync_copy(dst_r.at[t], dstb)
        pltpu.sync_copy(onesb, deg_s.at[pl.ds(r0, RPT)])
        plsc.subcore_barrier()

        def chunk(j, carry):
            pltpu.sync_copy(onesb.at[pl.ds(0, LANES)], deg_s.at[dstb.at[j]],
                            add=True)
            return carry

        lax.fori_loop(0, CH, chunk, None)
        plsc.subcore_barrier()

        @pl.when(cid == 0)
        def _():
            pltpu.sync_copy(deg_s.at[pl.ds(r0, RPT)], degc)
            pltpu.sync_copy(degc, out_r.at[pl.ds(r0, RPT)])

    return body(dst3, ones_rpt)


def _appnp_sc(src3, dst3, xda, d2a, sqb, zrows, F):
    mesh = plsc.VectorSubcoreMesh(core_axis_name="c", subcore_axis_name="s")
    CB = 64

    @functools.partial(
        pl.kernel,
        out_type=jax.ShapeDtypeStruct((2, NP, F), jnp.float32),
        mesh=mesh,
        scratch_types=[
            pltpu.VMEM_SHARED((NP, F), jnp.float32),
            pltpu.VMEM_SHARED((NP, F), jnp.float32),
            pltpu.VMEM((IDXB, LANES), jnp.int32),
            pltpu.VMEM((IDXB, LANES), jnp.int32),
            pltpu.VMEM((2, LANES, F), jnp.float32),
            pltpu.VMEM((CB, F), jnp.float32),
            pltpu.VMEM((CB, F), jnp.float32),
            pltpu.VMEM((CB, F), jnp.float32),
            pltpu.VMEM((CB, 16), jnp.float32),
            pltpu.VMEM((CB, 16), jnp.float32),
            pltpu.VMEM((CB, F), jnp.float32),
            pltpu.SemaphoreType.DMA((2,)),
            pltpu.SemaphoreType.DMA((2,)),
        ],
        compiler_params=pltpu.CompilerParams(use_tc_tiling_on_sc=False),
    )
    def body(src_r, dst_r, xd_r, d2_r, sq_r, z_r, out_r,
             T, Uc, srcb, dstb, gb, cT, cU, cX, cd, cs, zT, gs, ss):
        cid = lax.axis_index("c")
        t = lax.axis_index("s")
        r0 = t * RPT

        pltpu.sync_copy(z_r, zT)
        for cc in range(RPT // CB):
            rr = r0 + cc * CB
            pltpu.sync_copy(xd_r.at[cid, pl.ds(rr, CB)], cX)

            def urow(rI, carry):
                for c in range(F // 16):
                    sl = pl.ds(c * 16, 16)
                    cU[rI, sl] = cX[rI, sl] * (1.0 / ALPHA)
                return carry

            lax.fori_loop(0, CB, urow, None)
            pltpu.sync_copy(cU, Uc.at[pl.ds(rr, CB)])
            pltpu.sync_copy(zT, T.at[pl.ds(rr, CB)])
        plsc.subcore_barrier()

        def edge_phase():
            def drain(sem, p):
                pltpu.make_async_copy(xd_r.at[cid, pl.ds(0, LANES)],
                                      gb.at[p], sem.at[p]).wait()

            def blk(bI, carry):
                pltpu.sync_copy(src_r.at[t, pl.ds(bI * IDXB, IDXB)], srcb)
                pltpu.sync_copy(dst_r.at[t, pl.ds(bI * IDXB, IDXB)], dstb)
                pltpu.async_copy(Uc.at[srcb.at[0]], gb.at[0], gs.at[0])

                def chunk(j, c2):
                    p = lax.rem(j, 2)
                    pn = lax.rem(j + 1, 2)

                    @pl.when(j >= 1)
                    def _():
                        drain(ss, pn)

                    @pl.when(j + 1 < IDXB)
                    def _():
                        pltpu.async_copy(Uc.at[srcb.at[j + 1]], gb.at[pn],
                                         gs.at[pn])

                    drain(gs, p)
                    pltpu.async_copy(gb.at[p], T.at[dstb.at[j]], ss.at[p],
                                     add=True)
                    return c2

                lax.fori_loop(0, IDXB, chunk, None)
                drain(ss, 1)
                return carry

            lax.fori_loop(0, CH // IDXB, blk, None)
            plsc.subcore_barrier()

        def combine(is_final):
            for cc in range(RPT // CB):
                rr = r0 + cc * CB
                pltpu.sync_copy(T.at[pl.ds(rr, CB)], cT)
                pltpu.sync_copy(Uc.at[pl.ds(rr, CB)], cU)
                pltpu.sync_copy(xd_r.at[cid, pl.ds(rr, CB)], cX)
                pltpu.sync_copy(d2_r.at[pl.ds(rr, CB)], cd)
                if is_final:
                    pltpu.sync_copy(sq_r.at[pl.ds(rr, CB)], cs)

                def row(rI, carry):
                    bd = cd[rI, pl.ds(0, 16)]
                    for c in range(F // 16):
                        sl = pl.ds(c * 16, 16)
                        un = bd * (cT[rI, sl] + cU[rI, sl]) + cX[rI, sl]
                        if is_final:
                            un = un * cs[rI, pl.ds(0, 16)]
                        cU[rI, sl] = un
                    return carry

                lax.fori_loop(0, CB, row, None)
                if is_final:
                    pltpu.sync_copy(cU, out_r.at[cid, pl.ds(rr, CB)])
                else:
                    pltpu.sync_copy(cU, Uc.at[pl.ds(rr, CB)])
                    pltpu.sync_copy(zT, T.at[pl.ds(rr, CB)])

        def round_body(k, carry):
            edge_phase()
            combine(False)
            plsc.subcore_barrier()
            return carry

        lax.fori_loop(0, KPROP - 1, round_body, None)
        edge_phase()
        combine(True)

    return body(src3, dst3, xda, d2a, sqb, zrows)


def _tc_prep(x_p, W1, b1, deg2):
    def body(x_r, w_r, b_r, deg_r, xd_r, d2_r, sq_r):
        deg = deg_r[:, 0:1]
        dis = lax.rsqrt(deg)
        h = jnp.dot(x_r[...], w_r[...].T,
                    preferred_element_type=jnp.float32) + b_r[...][None, :]
        mask = lax.broadcasted_iota(jnp.int32, (NP, 1), 0) < N
        hd = jnp.where(mask, (ALPHA * dis) * h, 0.0)
        xd_r[0] = hd[:, :HID // 2]
        xd_r[1] = hd[:, HID // 2:]
        d2_r[...] = jnp.broadcast_to((1.0 - ALPHA) / deg, (NP, 16))
        sq_r[...] = jnp.broadcast_to(jnp.sqrt(deg), (NP, 16))

    return pl.pallas_call(
        body,
        out_shape=[
            jax.ShapeDtypeStruct((2, NP, HID // 2), jnp.float32),
            jax.ShapeDtypeStruct((NP, 16), jnp.float32),
            jax.ShapeDtypeStruct((NP, 16), jnp.float32),
        ],
    )(x_p, W1, b1, deg2)


def _tc_mid(H, g, be, W, b, d2b, sqb, Fo):
    def body(H_r, g_r, be_r, w_r, b_r, d2_r, sq_r, out_r):
        h = jnp.concatenate([H_r[0], H_r[1]], axis=1)
        mask = lax.broadcasted_iota(jnp.int32, (NP, 1), 0) < N
        hm = jnp.where(mask, h, 0.0)
        m = jnp.sum(hm, axis=0, keepdims=True) / N
        dcen = jnp.where(mask, h - m, 0.0)
        v = jnp.sum(dcen * dcen, axis=0, keepdims=True) / N
        hn = g_r[...][None, :] * (h - m) * lax.rsqrt(v + EPS) + be_r[...][None, :]
        hrelu = jnp.maximum(hn, 0.0)
        h2 = jnp.dot(hrelu, w_r[...].T,
                     preferred_element_type=jnp.float32) + b_r[...][None, :]
        adis = (ALPHA / (1.0 - ALPHA)) * d2_r[:, 0:1] * sq_r[:, 0:1]
        xd = jnp.where(mask, adis * h2, 0.0)
        out_r[0] = xd[:, :Fo // 2]
        out_r[1] = xd[:, Fo // 2:]

    return pl.pallas_call(
        body,
        out_shape=jax.ShapeDtypeStruct((2, NP, Fo // 2), jnp.float32),
    )(H, g, be, W, b, d2b, sqb)


def _tc_final(H, g, be):
    def body(H_r, g_r, be_r, out_r):
        h = jnp.concatenate([H_r[0], H_r[1]], axis=1)
        mask = lax.broadcasted_iota(jnp.int32, (NP, 1), 0) < N
        hm = jnp.where(mask, h, 0.0)
        m = jnp.sum(hm, axis=0, keepdims=True) / N
        dcen = jnp.where(mask, h - m, 0.0)
        v = jnp.sum(dcen * dcen, axis=0, keepdims=True) / N
        out_r[...] = g_r[...][None, :] * (h - m) * lax.rsqrt(v + EPS) + be_r[...][None, :]

    return pl.pallas_call(
        body,
        out_shape=jax.ShapeDtypeStruct((NP, NCLS), jnp.float32),
    )(H, g, be)


def kernel(x, edge_index, W1, b1, Wx, bx, W2, b2, g1, be1, g3, be3, g2, be2):
    ei = edge_index.astype(jnp.int32)
    src = ei[0]
    dst = ei[1]
    pad = EP - E
    src3 = jnp.concatenate([src, jnp.zeros((pad,), jnp.int32)]).reshape(NTILES, CH, LANES)
    dst3 = jnp.concatenate([dst, jnp.full((pad,), N, jnp.int32)]).reshape(NTILES, CH, LANES)
    x_p = jnp.pad(x, ((0, NP - N), (0, 0)))
    ones_rpt = jnp.ones((RPT, 16), jnp.float32)
    z64 = jnp.zeros((64, HID // 2), jnp.float32)
    z32 = jnp.zeros((64, NCLS // 2), jnp.float32)

    deg2 = _deg_sc(dst3, ones_rpt)
    xd1, d2b, sqb = _tc_prep(x_p, W1, b1, deg2)
    H1 = _appnp_sc(src3, dst3, xd1, d2b, sqb, z64, HID // 2)
    xd2 = _tc_mid(H1, g1, be1, Wx, bx, d2b, sqb, HID)
    H2 = _appnp_sc(src3, dst3, xd2, d2b, sqb, z64, HID // 2)
    xd3 = _tc_mid(H2, g3, be3, W2, b2, d2b, sqb, NCLS)
    H3 = _appnp_sc(src3, dst3, xd3, d2b, sqb, z32, NCLS // 2)
    out = _tc_final(H3, g2, be2)
    return out[:N]


# speedup vs baseline: 2.0080x; 1.0657x over previous
import functools

import jax
import jax.numpy as jnp
from jax import lax
from jax.experimental import pallas as pl
from jax.experimental.pallas import tpu as pltpu
from jax.experimental.pallas import tpu_sc as plsc

N = 10000
E = 320000
D_IN = 128
HID = 128
NCLS = 64
ALPHA = 0.1
KPROP = 10
EPS = 1e-5

NTILES = 16
LANES = 128
CH = 160
IDXB = 16
EP = NTILES * CH * LANES
NP = 10240
RPT = NP // NTILES


def _deg_sc(dst3, ones_rpt):
    mesh = plsc.VectorSubcoreMesh(core_axis_name="c", subcore_axis_name="s")

    @functools.partial(
        pl.kernel,
        out_type=jax.ShapeDtypeStruct((NP, 16), jnp.float32),
        mesh=mesh,
        scratch_types=[
            pltpu.VMEM_SHARED((NP, 16), jnp.float32),
            pltpu.VMEM((CH, LANES), jnp.int32),
            pltpu.VMEM((RPT, 16), jnp.float32),
            pltpu.VMEM((RPT, 16), jnp.float32),
        ],
        compiler_params=pltpu.CompilerParams(use_tc_tiling_on_sc=False),
    )
    def body(dst_r, ones_r, out_r, deg_s, dstb, onesb, degc):
        cid = lax.axis_index("c")
        t = lax.axis_index("s")
        r0 = t * RPT
        pltpu.sync_copy(ones_r, onesb)
        pltpu.sync_copy(dst_r.at[t], dstb)
        pltpu.sync_copy(onesb, deg_s.at[pl.ds(r0, RPT)])
        plsc.subcore_barrier()

        def chunk(j, carry):
            pltpu.sync_copy(onesb.at[pl.ds(0, LANES)], deg_s.at[dstb.at[j]],
                            add=True)
            return carry

        lax.fori_loop(0, CH, chunk, None)
        plsc.subcore_barrier()

        @pl.when(cid == 0)
        def _():
            pltpu.sync_copy(deg_s.at[pl.ds(r0, RPT)], degc)
            pltpu.sync_copy(degc, out_r.at[pl.ds(r0, RPT)])

    return body(dst3, ones_rpt)


def _appnp_sc(src3, dst3, xda, d2a, sqb, zrows, F):
    mesh = plsc.VectorSubcoreMesh(core_axis_name="c", subcore_axis_name="s")
    CB = 32

    @functools.partial(
        pl.kernel,
        out_type=jax.ShapeDtypeStruct((2, NP, F), jnp.float32),
        mesh=mesh,
        scratch_types=[
            pltpu.VMEM_SHARED((NP, F), jnp.float32),
            pltpu.VMEM_SHARED((NP, F), jnp.float32),
            pltpu.VMEM((IDXB, LANES), jnp.int32),
            pltpu.VMEM((IDXB, LANES), jnp.int32),
            pltpu.VMEM((3, LANES, F), jnp.float32),
            pltpu.VMEM((CB, F), jnp.float32),
            pltpu.VMEM((CB, F), jnp.float32),
            pltpu.VMEM((CB, F), jnp.float32),
            pltpu.VMEM((CB, 16), jnp.float32),
            pltpu.VMEM((CB, 16), jnp.float32),
            pltpu.VMEM((CB, F), jnp.float32),
            pltpu.SemaphoreType.DMA((3,)),
            pltpu.SemaphoreType.DMA((3,)),
        ],
        compiler_params=pltpu.CompilerParams(use_tc_tiling_on_sc=False),
    )
    def body(src_r, dst_r, xd_r, d2_r, sq_r, z_r, out_r,
             T, Uc, srcb, dstb, gb, cT, cU, cX, cd, cs, zT, gs, ss):
        cid = lax.axis_index("c")
        t = lax.axis_index("s")
        r0 = t * RPT

        pltpu.sync_copy(z_r, zT)
        for cc in range(RPT // CB):
            rr = r0 + cc * CB
            pltpu.sync_copy(xd_r.at[cid, pl.ds(rr, CB)], cX)

            def urow(rI, carry):
                for c in range(F // 16):
                    sl = pl.ds(c * 16, 16)
                    cU[rI, sl] = cX[rI, sl] * (1.0 / ALPHA)
                return carry

            lax.fori_loop(0, CB, urow, None)
            pltpu.sync_copy(cU, Uc.at[pl.ds(rr, CB)])
            pltpu.sync_copy(zT, T.at[pl.ds(rr, CB)])
        plsc.subcore_barrier()

        def edge_phase():
            def drain(sem, p):
                pltpu.make_async_copy(xd_r.at[cid, pl.ds(0, LANES)],
                                      gb.at[p], sem.at[p]).wait()

            def blk(bI, carry):
                pltpu.sync_copy(src_r.at[t, pl.ds(bI * IDXB, IDXB)], srcb)
                pltpu.sync_copy(dst_r.at[t, pl.ds(bI * IDXB, IDXB)], dstb)
                pltpu.async_copy(Uc.at[srcb.at[0]], gb.at[0], gs.at[0])

                def chunk(j, c2):
                    p = lax.rem(j, 3)
                    pn = lax.rem(j + 1, 3)

                    @pl.when(j >= 2)
                    def _():
                        drain(ss, pn)

                    @pl.when(j + 1 < IDXB)
                    def _():
                        pltpu.async_copy(Uc.at[srcb.at[j + 1]], gb.at[pn],
                                         gs.at[pn])

                    drain(gs, p)
                    pltpu.async_copy(gb.at[p], T.at[dstb.at[j]], ss.at[p],
                                     add=True)
                    return c2

                lax.fori_loop(0, IDXB, chunk, None)
                drain(ss, (IDXB - 2) % 3)
                drain(ss, (IDXB - 1) % 3)
                return carry

            lax.fori_loop(0, CH // IDXB, blk, None)
            plsc.subcore_barrier()

        def combine(is_final):
            for cc in range(RPT // CB):
                rr = r0 + cc * CB
                pltpu.sync_copy(T.at[pl.ds(rr, CB)], cT)
                pltpu.sync_copy(Uc.at[pl.ds(rr, CB)], cU)
                pltpu.sync_copy(xd_r.at[cid, pl.ds(rr, CB)], cX)
                pltpu.sync_copy(d2_r.at[pl.ds(rr, CB)], cd)
                if is_final:
                    pltpu.sync_copy(sq_r.at[pl.ds(rr, CB)], cs)

                def row(rI, carry):
                    bd = cd[rI, pl.ds(0, 16)]
                    for c in range(F // 16):
                        sl = pl.ds(c * 16, 16)
                        un = bd * (cT[rI, sl] + cU[rI, sl]) + cX[rI, sl]
                        if is_final:
                            un = un * cs[rI, pl.ds(0, 16)]
                        cU[rI, sl] = un
                    return carry

                lax.fori_loop(0, CB, row, None)
                if is_final:
                    pltpu.sync_copy(cU, out_r.at[cid, pl.ds(rr, CB)])
                else:
                    pltpu.sync_copy(cU, Uc.at[pl.ds(rr, CB)])
                    pltpu.sync_copy(zT, T.at[pl.ds(rr, CB)])

        def round_body(k, carry):
            edge_phase()
            combine(False)
            plsc.subcore_barrier()
            return carry

        lax.fori_loop(0, KPROP - 1, round_body, None)
        edge_phase()
        combine(True)

    return body(src3, dst3, xda, d2a, sqb, zrows)


def _tc_prep(x_p, W1, b1, deg2):
    def body(x_r, w_r, b_r, deg_r, xd_r, d2_r, sq_r):
        deg = deg_r[:, 0:1]
        dis = lax.rsqrt(deg)
        h = jnp.dot(x_r[...], w_r[...].T,
                    preferred_element_type=jnp.float32) + b_r[...][None, :]
        mask = lax.broadcasted_iota(jnp.int32, (NP, 1), 0) < N
        hd = jnp.where(mask, (ALPHA * dis) * h, 0.0)
        xd_r[0] = hd[:, :HID // 2]
        xd_r[1] = hd[:, HID // 2:]
        d2_r[...] = jnp.broadcast_to((1.0 - ALPHA) / deg, (NP, 16))
        sq_r[...] = jnp.broadcast_to(jnp.sqrt(deg), (NP, 16))

    return pl.pallas_call(
        body,
        out_shape=[
            jax.ShapeDtypeStruct((2, NP, HID // 2), jnp.float32),
            jax.ShapeDtypeStruct((NP, 16), jnp.float32),
            jax.ShapeDtypeStruct((NP, 16), jnp.float32),
        ],
    )(x_p, W1, b1, deg2)


def _tc_mid(H, g, be, W, b, d2b, sqb, Fo):
    def body(H_r, g_r, be_r, w_r, b_r, d2_r, sq_r, out_r):
        h = jnp.concatenate([H_r[0], H_r[1]], axis=1)
        mask = lax.broadcasted_iota(jnp.int32, (NP, 1), 0) < N
        hm = jnp.where(mask, h, 0.0)
        m = jnp.sum(hm, axis=0, keepdims=True) / N
        dcen = jnp.where(mask, h - m, 0.0)
        v = jnp.sum(dcen * dcen, axis=0, keepdims=True) / N
        hn = g_r[...][None, :] * (h - m) * lax.rsqrt(v + EPS) + be_r[...][None, :]
        hrelu = jnp.maximum(hn, 0.0)
        h2 = jnp.dot(hrelu, w_r[...].T,
                     preferred_element_type=jnp.float32) + b_r[...][None, :]
        adis = (ALPHA / (1.0 - ALPHA)) * d2_r[:, 0:1] * sq_r[:, 0:1]
        xd = jnp.where(mask, adis * h2, 0.0)
        out_r[0] = xd[:, :Fo // 2]
        out_r[1] = xd[:, Fo // 2:]

    return pl.pallas_call(
        body,
        out_shape=jax.ShapeDtypeStruct((2, NP, Fo // 2), jnp.float32),
    )(H, g, be, W, b, d2b, sqb)


def _tc_final(H, g, be):
    def body(H_r, g_r, be_r, out_r):
        h = jnp.concatenate([H_r[0], H_r[1]], axis=1)
        mask = lax.broadcasted_iota(jnp.int32, (NP, 1), 0) < N
        hm = jnp.where(mask, h, 0.0)
        m = jnp.sum(hm, axis=0, keepdims=True) / N
        dcen = jnp.where(mask, h - m, 0.0)
        v = jnp.sum(dcen * dcen, axis=0, keepdims=True) / N
        out_r[...] = g_r[...][None, :] * (h - m) * lax.rsqrt(v + EPS) + be_r[...][None, :]

    return pl.pallas_call(
        body,
        out_shape=jax.ShapeDtypeStruct((NP, NCLS), jnp.float32),
    )(H, g, be)


def kernel(x, edge_index, W1, b1, Wx, bx, W2, b2, g1, be1, g3, be3, g2, be2):
    ei = edge_index.astype(jnp.int32)
    src = ei[0]
    dst = ei[1]
    pad = EP - E
    src3 = jnp.concatenate([src, jnp.zeros((pad,), jnp.int32)]).reshape(NTILES, CH, LANES)
    dst3 = jnp.concatenate([dst, jnp.full((pad,), N, jnp.int32)]).reshape(NTILES, CH, LANES)
    x_p = jnp.pad(x, ((0, NP - N), (0, 0)))
    ones_rpt = jnp.ones((RPT, 16), jnp.float32)
    z64 = jnp.zeros((32, HID // 2), jnp.float32)
    z32 = jnp.zeros((32, NCLS // 2), jnp.float32)

    deg2 = _deg_sc(dst3, ones_rpt)
    xd1, d2b, sqb = _tc_prep(x_p, W1, b1, deg2)
    H1 = _appnp_sc(src3, dst3, xd1, d2b, sqb, z64, HID // 2)
    xd2 = _tc_mid(H1, g1, be1, Wx, bx, d2b, sqb, HID)
    H2 = _appnp_sc(src3, dst3, xd2, d2b, sqb, z64, HID // 2)
    xd3 = _tc_mid(H2, g3, be3, W2, b2, d2b, sqb, NCLS)
    H3 = _appnp_sc(src3, dst3, xd3, d2b, sqb, z32, NCLS // 2)
    out = _tc_final(H3, g2, be2)
    return out[:N]


# speedup vs baseline: 2.1515x; 1.0714x over previous
import functools

import jax
import jax.numpy as jnp
from jax import lax
from jax.experimental import pallas as pl
from jax.experimental.pallas import tpu as pltpu
from jax.experimental.pallas import tpu_sc as plsc

N = 10000
E = 320000
D_IN = 128
HID = 128
NCLS = 64
ALPHA = 0.1
KPROP = 10
EPS = 1e-5

NTILES = 16
LANES = 128
CH = 160
IDXB = 32
EP = NTILES * CH * LANES
NP = 10240
RPT = NP // NTILES


def _deg_sc(dst3, ones_rpt):
    mesh = plsc.VectorSubcoreMesh(core_axis_name="c", subcore_axis_name="s")

    @functools.partial(
        pl.kernel,
        out_type=jax.ShapeDtypeStruct((NP, 16), jnp.float32),
        mesh=mesh,
        scratch_types=[
            pltpu.VMEM_SHARED((NP, 16), jnp.float32),
            pltpu.VMEM((CH, LANES), jnp.int32),
            pltpu.VMEM((RPT, 16), jnp.float32),
            pltpu.VMEM((RPT, 16), jnp.float32),
        ],
        compiler_params=pltpu.CompilerParams(use_tc_tiling_on_sc=False),
    )
    def body(dst_r, ones_r, out_r, deg_s, dstb, onesb, degc):
        cid = lax.axis_index("c")
        t = lax.axis_index("s")
        r0 = t * RPT
        pltpu.sync_copy(ones_r, onesb)
        pltpu.sync_copy(dst_r.at[t], dstb)
        pltpu.sync_copy(onesb, deg_s.at[pl.ds(r0, RPT)])
        plsc.subcore_barrier()

        def chunk(j, carry):
            pltpu.sync_copy(onesb.at[pl.ds(0, LANES)], deg_s.at[dstb.at[j]],
                            add=True)
            return carry

        lax.fori_loop(0, CH, chunk, None)
        plsc.subcore_barrier()

        @pl.when(cid == 0)
        def _():
            pltpu.sync_copy(deg_s.at[pl.ds(r0, RPT)], degc)
            pltpu.sync_copy(degc, out_r.at[pl.ds(r0, RPT)])

    return body(dst3, ones_rpt)


def _appnp_sc(src3, dst3, xda, d2a, sqb, zrows, F):
    mesh = plsc.VectorSubcoreMesh(core_axis_name="c", subcore_axis_name="s")
    CB = 32

    @functools.partial(
        pl.kernel,
        out_type=jax.ShapeDtypeStruct((2, NP, F), jnp.float32),
        mesh=mesh,
        scratch_types=[
            pltpu.VMEM_SHARED((NP, F), jnp.float32),
            pltpu.VMEM_SHARED((NP, F), jnp.float32),
            pltpu.VMEM((IDXB, LANES), jnp.int32),
            pltpu.VMEM((IDXB, LANES), jnp.int32),
            pltpu.VMEM((3, LANES, F), jnp.float32),
            pltpu.VMEM((CB, F), jnp.float32),
            pltpu.VMEM((CB, F), jnp.float32),
            pltpu.VMEM((CB, F), jnp.float32),
            pltpu.VMEM((CB, 16), jnp.float32),
            pltpu.VMEM((CB, 16), jnp.float32),
            pltpu.VMEM((CB, F), jnp.float32),
            pltpu.SemaphoreType.DMA((3,)),
            pltpu.SemaphoreType.DMA((3,)),
        ],
        compiler_params=pltpu.CompilerParams(use_tc_tiling_on_sc=False),
    )
    def body(src_r, dst_r, xd_r, d2_r, sq_r, z_r, out_r,
             T, Uc, srcb, dstb, gb, cT, cU, cX, cd, cs, zT, gs, ss):
        cid = lax.axis_index("c")
        t = lax.axis_index("s")
        r0 = t * RPT

        pltpu.sync_copy(z_r, zT)
        for cc in range(RPT // CB):
            rr = r0 + cc * CB
            pltpu.sync_copy(xd_r.at[cid, pl.ds(rr, CB)], cX)

            def urow(rI, carry):
                for c in range(F // 16):
                    sl = pl.ds(c * 16, 16)
                    cU[rI, sl] = cX[rI, sl] * (1.0 / ALPHA)
                return carry

            lax.fori_loop(0, CB, urow, None)
            pltpu.sync_copy(cU, Uc.at[pl.ds(rr, CB)])
            pltpu.sync_copy(zT, T.at[pl.ds(rr, CB)])
        plsc.subcore_barrier()

        def edge_phase():
            def drain(sem, p):
                pltpu.make_async_copy(xd_r.at[cid, pl.ds(0, LANES)],
                                      gb.at[p], sem.at[p]).wait()

            def blk(bI, carry):
                pltpu.sync_copy(src_r.at[t, pl.ds(bI * IDXB, IDXB)], srcb)
                pltpu.sync_copy(dst_r.at[t, pl.ds(bI * IDXB, IDXB)], dstb)
                pltpu.async_copy(Uc.at[srcb.at[0]], gb.at[0], gs.at[0])

                def chunk(j, c2):
                    p = lax.rem(j, 3)
                    pn = lax.rem(j + 1, 3)

                    @pl.when(j >= 2)
                    def _():
                        drain(ss, pn)

                    @pl.when(j + 1 < IDXB)
                    def _():
                        pltpu.async_copy(Uc.at[srcb.at[j + 1]], gb.at[pn],
                                         gs.at[pn])

                    drain(gs, p)
                    pltpu.async_copy(gb.at[p], T.at[dstb.at[j]], ss.at[p],
                                     add=True)
                    return c2

                lax.fori_loop(0, IDXB, chunk, None)
                drain(ss, (IDXB - 2) % 3)
                drain(ss, (IDXB - 1) % 3)
                return carry

            lax.fori_loop(0, CH // IDXB, blk, None)
            plsc.subcore_barrier()

        def combine(is_final):
            for cc in range(RPT // CB):
                rr = r0 + cc * CB
                pltpu.sync_copy(T.at[pl.ds(rr, CB)], cT)
                pltpu.sync_copy(Uc.at[pl.ds(rr, CB)], cU)
                pltpu.sync_copy(xd_r.at[cid, pl.ds(rr, CB)], cX)
                pltpu.sync_copy(d2_r.at[pl.ds(rr, CB)], cd)
                if is_final:
                    pltpu.sync_copy(sq_r.at[pl.ds(rr, CB)], cs)

                def row(rI, carry):
                    bd = cd[rI, pl.ds(0, 16)]
                    for c in range(F // 16):
                        sl = pl.ds(c * 16, 16)
                        un = bd * (cT[rI, sl] + cU[rI, sl]) + cX[rI, sl]
                        if is_final:
                            un = un * cs[rI, pl.ds(0, 16)]
                        cU[rI, sl] = un
                    return carry

                lax.fori_loop(0, CB, row, None)
                if is_final:
                    pltpu.sync_copy(cU, out_r.at[cid, pl.ds(rr, CB)])
                else:
                    pltpu.sync_copy(cU, Uc.at[pl.ds(rr, CB)])
                    pltpu.sync_copy(zT, T.at[pl.ds(rr, CB)])

        def round_body(k, carry):
            edge_phase()
            combine(False)
            plsc.subcore_barrier()
            return carry

        lax.fori_loop(0, KPROP - 1, round_body, None)
        edge_phase()
        combine(True)

    return body(src3, dst3, xda, d2a, sqb, zrows)


def _tc_prep(x_p, W1, b1, deg2):
    def body(x_r, w_r, b_r, deg_r, xd_r, d2_r, sq_r):
        deg = deg_r[:, 0:1]
        dis = lax.rsqrt(deg)
        h = jnp.dot(x_r[...], w_r[...].T,
                    preferred_element_type=jnp.float32) + b_r[...][None, :]
        mask = lax.broadcasted_iota(jnp.int32, (NP, 1), 0) < N
        hd = jnp.where(mask, (ALPHA * dis) * h, 0.0)
        xd_r[0] = hd[:, :HID // 2]
        xd_r[1] = hd[:, HID // 2:]
        d2_r[...] = jnp.broadcast_to((1.0 - ALPHA) / deg, (NP, 16))
        sq_r[...] = jnp.broadcast_to(jnp.sqrt(deg), (NP, 16))

    return pl.pallas_call(
        body,
        out_shape=[
            jax.ShapeDtypeStruct((2, NP, HID // 2), jnp.float32),
            jax.ShapeDtypeStruct((NP, 16), jnp.float32),
            jax.ShapeDtypeStruct((NP, 16), jnp.float32),
        ],
    )(x_p, W1, b1, deg2)


def _tc_mid(H, g, be, W, b, d2b, sqb, Fo):
    def body(H_r, g_r, be_r, w_r, b_r, d2_r, sq_r, out_r):
        h = jnp.concatenate([H_r[0], H_r[1]], axis=1)
        mask = lax.broadcasted_iota(jnp.int32, (NP, 1), 0) < N
        hm = jnp.where(mask, h, 0.0)
        m = jnp.sum(hm, axis=0, keepdims=True) / N
        dcen = jnp.where(mask, h - m, 0.0)
        v = jnp.sum(dcen * dcen, axis=0, keepdims=True) / N
        hn = g_r[...][None, :] * (h - m) * lax.rsqrt(v + EPS) + be_r[...][None, :]
        hrelu = jnp.maximum(hn, 0.0)
        h2 = jnp.dot(hrelu, w_r[...].T,
                     preferred_element_type=jnp.float32) + b_r[...][None, :]
        adis = (ALPHA / (1.0 - ALPHA)) * d2_r[:, 0:1] * sq_r[:, 0:1]
        xd = jnp.where(mask, adis * h2, 0.0)
        out_r[0] = xd[:, :Fo // 2]
        out_r[1] = xd[:, Fo // 2:]

    return pl.pallas_call(
        body,
        out_shape=jax.ShapeDtypeStruct((2, NP, Fo // 2), jnp.float32),
    )(H, g, be, W, b, d2b, sqb)


def _tc_final(H, g, be):
    def body(H_r, g_r, be_r, out_r):
        h = jnp.concatenate([H_r[0], H_r[1]], axis=1)
        mask = lax.broadcasted_iota(jnp.int32, (NP, 1), 0) < N
        hm = jnp.where(mask, h, 0.0)
        m = jnp.sum(hm, axis=0, keepdims=True) / N
        dcen = jnp.where(mask, h - m, 0.0)
        v = jnp.sum(dcen * dcen, axis=0, keepdims=True) / N
        out_r[...] = g_r[...][None, :] * (h - m) * lax.rsqrt(v + EPS) + be_r[...][None, :]

    return pl.pallas_call(
        body,
        out_shape=jax.ShapeDtypeStruct((NP, NCLS), jnp.float32),
    )(H, g, be)


def kernel(x, edge_index, W1, b1, Wx, bx, W2, b2, g1, be1, g3, be3, g2, be2):
    ei = edge_index.astype(jnp.int32)
    src = ei[0]
    dst = ei[1]
    pad = EP - E
    src3 = jnp.concatenate([src, jnp.zeros((pad,), jnp.int32)]).reshape(NTILES, CH, LANES)
    dst3 = jnp.concatenate([dst, jnp.full((pad,), N, jnp.int32)]).reshape(NTILES, CH, LANES)
    x_p = jnp.pad(x, ((0, NP - N), (0, 0)))
    ones_rpt = jnp.ones((RPT, 16), jnp.float32)
    z64 = jnp.zeros((32, HID // 2), jnp.float32)
    z32 = jnp.zeros((32, NCLS // 2), jnp.float32)

    deg2 = _deg_sc(dst3, ones_rpt)
    xd1, d2b, sqb = _tc_prep(x_p, W1, b1, deg2)
    H1 = _appnp_sc(src3, dst3, xd1, d2b, sqb, z64, HID // 2)
    xd2 = _tc_mid(H1, g1, be1, Wx, bx, d2b, sqb, HID)
    H2 = _appnp_sc(src3, dst3, xd2, d2b, sqb, z64, HID // 2)
    xd3 = _tc_mid(H2, g3, be3, W2, b2, d2b, sqb, NCLS)
    H3 = _appnp_sc(src3, dst3, xd3, d2b, sqb, z32, NCLS // 2)
    out = _tc_final(H3, g2, be2)
    return out[:N]


# speedup vs baseline: 2.1720x; 1.0095x over previous
import functools

import jax
import jax.numpy as jnp
from jax import lax
from jax.experimental import pallas as pl
from jax.experimental.pallas import tpu as pltpu
from jax.experimental.pallas import tpu_sc as plsc

N = 10000
E = 320000
D_IN = 128
HID = 128
NCLS = 64
ALPHA = 0.1
KPROP = 10
EPS = 1e-5

NTILES = 16
LANES = 128
CH = 160
IDXB = 40
EP = NTILES * CH * LANES
NP = 10240
RPT = NP // NTILES


def _deg_sc(dst3, ones_rpt):
    mesh = plsc.VectorSubcoreMesh(core_axis_name="c", subcore_axis_name="s")

    @functools.partial(
        pl.kernel,
        out_type=jax.ShapeDtypeStruct((NP, 16), jnp.float32),
        mesh=mesh,
        scratch_types=[
            pltpu.VMEM_SHARED((NP, 16), jnp.float32),
            pltpu.VMEM((CH, LANES), jnp.int32),
            pltpu.VMEM((RPT, 16), jnp.float32),
            pltpu.VMEM((RPT, 16), jnp.float32),
        ],
        compiler_params=pltpu.CompilerParams(use_tc_tiling_on_sc=False),
    )
    def body(dst_r, ones_r, out_r, deg_s, dstb, onesb, degc):
        cid = lax.axis_index("c")
        t = lax.axis_index("s")
        r0 = t * RPT
        pltpu.sync_copy(ones_r, onesb)
        pltpu.sync_copy(dst_r.at[t], dstb)
        pltpu.sync_copy(onesb, deg_s.at[pl.ds(r0, RPT)])
        plsc.subcore_barrier()

        def chunk(j, carry):
            pltpu.sync_copy(onesb.at[pl.ds(0, LANES)], deg_s.at[dstb.at[j]],
                            add=True)
            return carry

        lax.fori_loop(0, CH, chunk, None)
        plsc.subcore_barrier()

        @pl.when(cid == 0)
        def _():
            pltpu.sync_copy(deg_s.at[pl.ds(r0, RPT)], degc)
            pltpu.sync_copy(degc, out_r.at[pl.ds(r0, RPT)])

    return body(dst3, ones_rpt)


def _appnp_sc(src3, dst3, xda, d2a, sqb, zrows, F):
    mesh = plsc.VectorSubcoreMesh(core_axis_name="c", subcore_axis_name="s")
    CB = 32

    @functools.partial(
        pl.kernel,
        out_type=jax.ShapeDtypeStruct((2, NP, F), jnp.float32),
        mesh=mesh,
        scratch_types=[
            pltpu.VMEM_SHARED((NP, F), jnp.float32),
            pltpu.VMEM_SHARED((NP, F), jnp.float32),
            pltpu.VMEM((IDXB, LANES), jnp.int32),
            pltpu.VMEM((IDXB, LANES), jnp.int32),
            pltpu.VMEM((3, LANES, F), jnp.float32),
            pltpu.VMEM((CB, F), jnp.float32),
            pltpu.VMEM((CB, F), jnp.float32),
            pltpu.VMEM((CB, F), jnp.float32),
            pltpu.VMEM((CB, 16), jnp.float32),
            pltpu.VMEM((CB, 16), jnp.float32),
            pltpu.VMEM((CB, F), jnp.float32),
            pltpu.SemaphoreType.DMA((3,)),
            pltpu.SemaphoreType.DMA((3,)),
        ],
        compiler_params=pltpu.CompilerParams(use_tc_tiling_on_sc=False),
    )
    def body(src_r, dst_r, xd_r, d2_r, sq_r, z_r, out_r,
             T, Uc, srcb, dstb, gb, cT, cU, cX, cd, cs, zT, gs, ss):
        cid = lax.axis_index("c")
        t = lax.axis_index("s")
        r0 = t * RPT

        pltpu.sync_copy(z_r, zT)
        for cc in range(RPT // CB):
            rr = r0 + cc * CB
            pltpu.sync_copy(xd_r.at[cid, pl.ds(rr, CB)], cX)

            def urow(rI, carry):
                for c in range(F // 16):
                    sl = pl.ds(c * 16, 16)
                    cU[rI, sl] = cX[rI, sl] * (1.0 / ALPHA)
                return carry

            lax.fori_loop(0, CB, urow, None)
            pltpu.sync_copy(cU, Uc.at[pl.ds(rr, CB)])
            pltpu.sync_copy(zT, T.at[pl.ds(rr, CB)])
        plsc.subcore_barrier()

        def edge_phase():
            def drain(sem, p):
                pltpu.make_async_copy(xd_r.at[cid, pl.ds(0, LANES)],
                                      gb.at[p], sem.at[p]).wait()

            def blk(bI, carry):
                pltpu.sync_copy(src_r.at[t, pl.ds(bI * IDXB, IDXB)], srcb)
                pltpu.sync_copy(dst_r.at[t, pl.ds(bI * IDXB, IDXB)], dstb)
                pltpu.async_copy(Uc.at[srcb.at[0]], gb.at[0], gs.at[0])

                def chunk(j, c2):
                    p = lax.rem(j, 3)
                    pn = lax.rem(j + 1, 3)

                    @pl.when(j >= 2)
                    def _():
                        drain(ss, pn)

                    @pl.when(j + 1 < IDXB)
                    def _():
                        pltpu.async_copy(Uc.at[srcb.at[j + 1]], gb.at[pn],
                                         gs.at[pn])

                    drain(gs, p)
                    pltpu.async_copy(gb.at[p], T.at[dstb.at[j]], ss.at[p],
                                     add=True)
                    return c2

                lax.fori_loop(0, IDXB, chunk, None)
                drain(ss, (IDXB - 2) % 3)
                drain(ss, (IDXB - 1) % 3)
                return carry

            lax.fori_loop(0, CH // IDXB, blk, None)
            plsc.subcore_barrier()

        def combine(is_final):
            for cc in range(RPT // CB):
                rr = r0 + cc * CB
                pltpu.sync_copy(T.at[pl.ds(rr, CB)], cT)
                pltpu.sync_copy(Uc.at[pl.ds(rr, CB)], cU)
                pltpu.sync_copy(xd_r.at[cid, pl.ds(rr, CB)], cX)
                pltpu.sync_copy(d2_r.at[pl.ds(rr, CB)], cd)
                if is_final:
                    pltpu.sync_copy(sq_r.at[pl.ds(rr, CB)], cs)

                def row(rI, carry):
                    bd = cd[rI, pl.ds(0, 16)]
                    for c in range(F // 16):
                        sl = pl.ds(c * 16, 16)
                        un = bd * (cT[rI, sl] + cU[rI, sl]) + cX[rI, sl]
                        if is_final:
                            un = un * cs[rI, pl.ds(0, 16)]
                        cU[rI, sl] = un
                    return carry

                lax.fori_loop(0, CB, row, None)
                if is_final:
                    pltpu.sync_copy(cU, out_r.at[cid, pl.ds(rr, CB)])
                else:
                    pltpu.sync_copy(cU, Uc.at[pl.ds(rr, CB)])
                    pltpu.sync_copy(zT, T.at[pl.ds(rr, CB)])

        def round_body(k, carry):
            edge_phase()
            combine(False)
            plsc.subcore_barrier()
            return carry

        lax.fori_loop(0, KPROP - 1, round_body, None)
        edge_phase()
        combine(True)

    return body(src3, dst3, xda, d2a, sqb, zrows)


def _tc_prep(x_p, W1, b1, deg2):
    def body(x_r, w_r, b_r, deg_r, xd_r, d2_r, sq_r):
        deg = deg_r[:, 0:1]
        dis = lax.rsqrt(deg)
        h = jnp.dot(x_r[...], w_r[...].T,
                    preferred_element_type=jnp.float32) + b_r[...][None, :]
        mask = lax.broadcasted_iota(jnp.int32, (NP, 1), 0) < N
        hd = jnp.where(mask, (ALPHA * dis) * h, 0.0)
        xd_r[0] = hd[:, :HID // 2]
        xd_r[1] = hd[:, HID // 2:]
        d2_r[...] = jnp.broadcast_to((1.0 - ALPHA) / deg, (NP, 16))
        sq_r[...] = jnp.broadcast_to(jnp.sqrt(deg), (NP, 16))

    return pl.pallas_call(
        body,
        out_shape=[
            jax.ShapeDtypeStruct((2, NP, HID // 2), jnp.float32),
            jax.ShapeDtypeStruct((NP, 16), jnp.float32),
            jax.ShapeDtypeStruct((NP, 16), jnp.float32),
        ],
    )(x_p, W1, b1, deg2)


def _tc_mid(H, g, be, W, b, d2b, sqb, Fo):
    def body(H_r, g_r, be_r, w_r, b_r, d2_r, sq_r, out_r):
        h = jnp.concatenate([H_r[0], H_r[1]], axis=1)
        mask = lax.broadcasted_iota(jnp.int32, (NP, 1), 0) < N
        hm = jnp.where(mask, h, 0.0)
        m = jnp.sum(hm, axis=0, keepdims=True) / N
        dcen = jnp.where(mask, h - m, 0.0)
        v = jnp.sum(dcen * dcen, axis=0, keepdims=True) / N
        hn = g_r[...][None, :] * (h - m) * lax.rsqrt(v + EPS) + be_r[...][None, :]
        hrelu = jnp.maximum(hn, 0.0)
        h2 = jnp.dot(hrelu, w_r[...].T,
                     preferred_element_type=jnp.float32) + b_r[...][None, :]
        adis = (ALPHA / (1.0 - ALPHA)) * d2_r[:, 0:1] * sq_r[:, 0:1]
        xd = jnp.where(mask, adis * h2, 0.0)
        out_r[0] = xd[:, :Fo // 2]
        out_r[1] = xd[:, Fo // 2:]

    return pl.pallas_call(
        body,
        out_shape=jax.ShapeDtypeStruct((2, NP, Fo // 2), jnp.float32),
    )(H, g, be, W, b, d2b, sqb)


def _tc_final(H, g, be):
    def body(H_r, g_r, be_r, out_r):
        h = jnp.concatenate([H_r[0], H_r[1]], axis=1)
        mask = lax.broadcasted_iota(jnp.int32, (NP, 1), 0) < N
        hm = jnp.where(mask, h, 0.0)
        m = jnp.sum(hm, axis=0, keepdims=True) / N
        dcen = jnp.where(mask, h - m, 0.0)
        v = jnp.sum(dcen * dcen, axis=0, keepdims=True) / N
        out_r[...] = g_r[...][None, :] * (h - m) * lax.rsqrt(v + EPS) + be_r[...][None, :]

    return pl.pallas_call(
        body,
        out_shape=jax.ShapeDtypeStruct((NP, NCLS), jnp.float32),
    )(H, g, be)


def kernel(x, edge_index, W1, b1, Wx, bx, W2, b2, g1, be1, g3, be3, g2, be2):
    ei = edge_index.astype(jnp.int32)
    src = ei[0]
    dst = ei[1]
    pad = EP - E
    src3 = jnp.concatenate([src, jnp.zeros((pad,), jnp.int32)]).reshape(NTILES, CH, LANES)
    dst3 = jnp.concatenate([dst, jnp.full((pad,), N, jnp.int32)]).reshape(NTILES, CH, LANES)
    x_p = jnp.pad(x, ((0, NP - N), (0, 0)))
    ones_rpt = jnp.ones((RPT, 16), jnp.float32)
    z64 = jnp.zeros((32, HID // 2), jnp.float32)
    z32 = jnp.zeros((32, NCLS // 2), jnp.float32)

    deg2 = _deg_sc(dst3, ones_rpt)
    xd1, d2b, sqb = _tc_prep(x_p, W1, b1, deg2)
    H1 = _appnp_sc(src3, dst3, xd1, d2b, sqb, z64, HID // 2)
    xd2 = _tc_mid(H1, g1, be1, Wx, bx, d2b, sqb, HID)
    H2 = _appnp_sc(src3, dst3, xd2, d2b, sqb, z64, HID // 2)
    xd3 = _tc_mid(H2, g3, be3, W2, b2, d2b, sqb, NCLS)
    H3 = _appnp_sc(src3, dst3, xd3, d2b, sqb, z32, NCLS // 2)
    out = _tc_final(H3, g2, be2)
    return out[:N]
